# baseline (speedup 1.0000x reference)
import jax
import jax.numpy as jnp
from jax.experimental import pallas as pl

BATCH = 32
N = 512
DIM = 128
INPUT_DIM = 64
TOPK = 20
EPS = 1e-5
NEG = -1e30
BSUB = 8

_INTERPRET = False


def _bf16_dot(a, b):
    return jax.lax.dot_general(
        a.astype(jnp.bfloat16), b.astype(jnp.bfloat16),
        (((1,), (0,)), ((), ())), preferred_element_type=jnp.float32)


def _split3_dot(a, b):
    a_hi = a.astype(jnp.bfloat16)
    a_lo = (a - a_hi.astype(jnp.float32)).astype(jnp.bfloat16)
    b_hi = b.astype(jnp.bfloat16)
    b_lo = (b - b_hi.astype(jnp.float32)).astype(jnp.bfloat16)
    dot = lambda p, q: jax.lax.dot_general(
        p, q, (((1,), (0,)), ((), ())), preferred_element_type=jnp.float32)
    return dot(a_hi, b_hi) + (dot(a_hi, b_lo) + dot(a_lo, b_hi))


def _graph_kernel(emb_ref, att_em_i_ref, att_em_j_ref,
                  bias_ref, embi_ref, embj_ref):
    emb = emb_ref[:]
    embb = emb.astype(jnp.bfloat16)
    gram = jax.lax.dot_general(embb, embb, (((1,), (1,)), ((), ())),
                               preferred_element_type=jnp.float32)
    nrm_col = jnp.sqrt(jnp.sum(emb * emb, axis=1, keepdims=True))
    nrm_row = nrm_col.T
    cos = gram / (nrm_col * nrm_row)
    lane = jax.lax.broadcasted_iota(jnp.int32, (N, N), 1)
    sub = jax.lax.broadcasted_iota(jnp.int32, (N, N), 0)

    def body(_, carry):
        c, m = carry
        rowmax = jnp.max(c, axis=1, keepdims=True)
        ismax = c == rowmax
        idx = jnp.min(jnp.where(ismax, lane, N), axis=1, keepdims=True)
        onehot = lane == idx
        m = jnp.where(onehot, 0.0, m)
        c = jnp.where(onehot, -2.0, c)
        return c, m

    diag = sub == lane
    m0 = jnp.where(diag, 0.0, NEG)
    c0 = jnp.where(diag, -2.0, cos)
    _, m = jax.lax.fori_loop(0, TOPK - 1, body, (c0, m0))
    bias_ref[:] = m
    embi_ref[:] = jnp.sum(emb * att_em_i_ref[:], axis=1, keepdims=True)
    embj_col = jnp.sum(emb * att_em_j_ref[:], axis=1, keepdims=True)
    embj_ref[:] = embj_col.T


def _attn_kernel(data_ref, lin_w_ref, bias_ref, embi_ref, embj_ref,
                 att_i_ref, att_j_ref, out_ref):
    for i in range(BSUB):
        xb = data_ref[i]
        xl = _bf16_dot(xb, lin_w_ref[:])
        ai_col = jnp.sum(xl * att_i_ref[:], axis=1, keepdims=True) + embi_ref[:]
        aj_col = jnp.sum(xl * att_j_ref[:], axis=1, keepdims=True)
        aj_row = aj_col.T + embj_ref[:]
        alpha = ai_col + aj_row
        alpha = jnp.maximum(alpha, 0.2 * alpha) + bias_ref[:]
        amax = jnp.max(alpha, axis=1, keepdims=True)
        e = jnp.exp(alpha - amax)
        den = jnp.sum(e, axis=1, keepdims=True)
        agg = _split3_dot(e, xl)
        out_ref[i] = agg * (1.0 / (den + 1e-16))


def _bnout_kernel(out_ref, emb_ref, gnn_bias_ref, bn1_g_ref, bn1_b_ref,
                  bn2_g_ref, bn2_b_ref, out_w_ref, res_ref):
    o = out_ref[:] + gnn_bias_ref[:]
    inv = 1.0 / (BATCH * N)
    mu = jnp.sum(o, axis=0, keepdims=True) * inv
    var = jnp.sum(o * o, axis=0, keepdims=True) * inv - mu * mu
    o = (o - mu) / jnp.sqrt(var + EPS) * bn1_g_ref[:] + bn1_b_ref[:]
    o = jnp.maximum(o, 0.0)
    o = (o.reshape(BATCH, N, DIM) * emb_ref[:][None]).reshape(BATCH * N, DIM)
    mu2 = jnp.sum(o, axis=0, keepdims=True) * inv
    var2 = jnp.sum(o * o, axis=0, keepdims=True) * inv - mu2 * mu2
    o = (o - mu2) / jnp.sqrt(var2 + EPS) * bn2_g_ref[:] + bn2_b_ref[:]
    o = jnp.maximum(o, 0.0)
    res_ref[:] = _bf16_dot(o, out_w_ref[:])


def kernel(data, org_edge_index, emb, lin_w, att_i, att_j, att_em_i, att_em_j,
           gnn_bias, bn1_g, bn1_b, bn2_g, bn2_b, out_w, out_b):
    del org_edge_index
    f32 = jnp.float32
    bias, embi, embj = pl.pallas_call(
        _graph_kernel,
        out_shape=(jax.ShapeDtypeStruct((N, N), f32),
                   jax.ShapeDtypeStruct((N, 1), f32),
                   jax.ShapeDtypeStruct((1, N), f32)),
        interpret=_INTERPRET,
    )(emb, att_em_i.reshape(1, DIM), att_em_j.reshape(1, DIM))

    full = lambda shape: pl.BlockSpec(shape, lambda b: (0,) * len(shape))
    out = pl.pallas_call(
        _attn_kernel,
        grid=(BATCH // BSUB,),
        in_specs=[
            pl.BlockSpec((BSUB, N, INPUT_DIM), lambda b: (b, 0, 0)),
            full((INPUT_DIM, DIM)),
            full((N, N)),
            full((N, 1)),
            full((1, N)),
            full((1, DIM)),
            full((1, DIM)),
        ],
        out_specs=pl.BlockSpec((BSUB, N, DIM), lambda b: (b, 0, 0)),
        out_shape=jax.ShapeDtypeStruct((BATCH, N, DIM), f32),
        interpret=_INTERPRET,
    )(data, lin_w, bias, embi, embj,
      att_i.reshape(1, DIM), att_j.reshape(1, DIM))

    res = pl.pallas_call(
        _bnout_kernel,
        out_shape=jax.ShapeDtypeStruct((BATCH * N, 1), f32),
        interpret=_INTERPRET,
    )(out.reshape(BATCH * N, DIM), emb, gnn_bias.reshape(1, DIM),
      bn1_g.reshape(1, DIM), bn1_b.reshape(1, DIM),
      bn2_g.reshape(1, DIM), bn2_b.reshape(1, DIM), out_w)

    return (res + out_b).reshape(BATCH, N)


# speedup vs baseline: 1.1074x; 1.1074x over previous
import jax
import jax.numpy as jnp
from jax.experimental import pallas as pl
from jax.experimental.pallas import tpu as pltpu

BATCH = 32
N = 512
DIM = 128
INPUT_DIM = 64
TOPK = 20
EPS = 1e-5
NEG = -1e30
BSUB = 4
NSTEP = BATCH // BSUB

_INTERPRET = False


def _bf16_dot(a, b):
    return jax.lax.dot_general(
        a.astype(jnp.bfloat16), b.astype(jnp.bfloat16),
        (((1,), (0,)), ((), ())), preferred_element_type=jnp.float32)


def _split3_dot(a, b):
    a_hi = a.astype(jnp.bfloat16)
    a_lo = (a - a_hi.astype(jnp.float32)).astype(jnp.bfloat16)
    b_hi = b.astype(jnp.bfloat16)
    b_lo = (b - b_hi.astype(jnp.float32)).astype(jnp.bfloat16)
    dot = lambda p, q: jax.lax.dot_general(
        p, q, (((1,), (0,)), ((), ())), preferred_element_type=jnp.float32)
    return dot(a_hi, b_hi) + (dot(a_hi, b_lo) + dot(a_lo, b_hi))


def _fused_kernel(data_ref, lin_w_ref, emb_ref, att_i_ref, att_j_ref,
                  att_em_i_ref, att_em_j_ref, gnn_bias_ref, bn1_g_ref,
                  bn1_b_ref, bn2_g_ref, bn2_b_ref, out_w_ref,
                  res_ref,
                  bias_s, embi_s, embj_s, out_s, stats_s):
    s = pl.program_id(0)

    @pl.when(s == 0)
    def _graph():
        emb = emb_ref[:]
        embb = emb.astype(jnp.bfloat16)
        gram = jax.lax.dot_general(embb, embb, (((1,), (1,)), ((), ())),
                                   preferred_element_type=jnp.float32)
        nrm_col = jnp.sqrt(jnp.sum(emb * emb, axis=1, keepdims=True))
        cos = gram / (nrm_col * nrm_col.T)
        lane = jax.lax.broadcasted_iota(jnp.int32, (N, N), 1)
        sub = jax.lax.broadcasted_iota(jnp.int32, (N, N), 0)

        def body(_, carry):
            c, m = carry
            rowmax = jnp.max(c, axis=1, keepdims=True)
            ismax = c == rowmax
            idx = jnp.min(jnp.where(ismax, lane, N), axis=1, keepdims=True)
            onehot = lane == idx
            m = jnp.where(onehot, 0.0, m)
            c = jnp.where(onehot, -2.0, c)
            return c, m

        diag = sub == lane
        m0 = jnp.where(diag, 0.0, NEG)
        c0 = jnp.where(diag, -2.0, cos)
        _, m = jax.lax.fori_loop(0, TOPK - 1, body, (c0, m0))
        bias_s[:] = m
        embi_s[:] = jnp.sum(emb * att_em_i_ref[:], axis=1, keepdims=True)
        embj_s[:] = jnp.sum(emb * att_em_j_ref[:], axis=1, keepdims=True).T
        stats_s[:] = jnp.zeros((8, DIM), jnp.float32)

    @pl.when((s >= 1) & (s <= NSTEP))
    def _attn():
        acc0 = stats_s[0:1]
        acc1 = stats_s[1:2]
        for i in range(BSUB):
            xb = data_ref[i]
            xl = _bf16_dot(xb, lin_w_ref[:])
            ai_col = (jnp.sum(xl * att_i_ref[:], axis=1, keepdims=True)
                      + embi_s[:])
            aj_col = jnp.sum(xl * att_j_ref[:], axis=1, keepdims=True)
            alpha = ai_col + (aj_col.T + embj_s[:])
            alpha = jnp.maximum(alpha, 0.2 * alpha) + bias_s[:]
            amax = jnp.max(alpha, axis=1, keepdims=True)
            e = jnp.exp(alpha - amax)
            den = jnp.sum(e, axis=1, keepdims=True)
            agg = _split3_dot(e, xl) * (1.0 / (den + 1e-16))
            o = agg + gnn_bias_ref[:]
            out_s[pl.ds((s - 1) * (BSUB * N) + i * N, N), :] = o
            acc0 = acc0 + jnp.sum(o, axis=0, keepdims=True)
            acc1 = acc1 + jnp.sum(o * o, axis=0, keepdims=True)
        stats_s[0:1] = acc0
        stats_s[1:2] = acc1

    @pl.when(s == NSTEP + 1)
    def _bnout():
        inv = 1.0 / (BATCH * N)
        mu = stats_s[0:1] * inv
        var = stats_s[1:2] * inv - mu * mu
        o = out_s[:]
        o = (o - mu) / jnp.sqrt(var + EPS) * bn1_g_ref[:] + bn1_b_ref[:]
        o = jnp.maximum(o, 0.0)
        o = (o.reshape(BATCH, N, DIM) * emb_ref[:][None]).reshape(BATCH * N, DIM)
        mu2 = jnp.sum(o, axis=0, keepdims=True) * inv
        var2 = jnp.sum(o * o, axis=0, keepdims=True) * inv - mu2 * mu2
        o = (o - mu2) / jnp.sqrt(var2 + EPS) * bn2_g_ref[:] + bn2_b_ref[:]
        o = jnp.maximum(o, 0.0)
        res_ref[:] = _bf16_dot(o, out_w_ref[:])


def kernel(data, org_edge_index, emb, lin_w, att_i, att_j, att_em_i, att_em_j,
           gnn_bias, bn1_g, bn1_b, bn2_g, bn2_b, out_w, out_b):
    del org_edge_index
    f32 = jnp.float32
    full = lambda shape: pl.BlockSpec(shape, lambda s: (0,) * len(shape))
    vec = full((1, DIM))
    res = pl.pallas_call(
        _fused_kernel,
        grid=(NSTEP + 2,),
        in_specs=[
            pl.BlockSpec((BSUB, N, INPUT_DIM),
                         lambda s: (jnp.clip(s - 1, 0, NSTEP - 1), 0, 0)),
            full((INPUT_DIM, DIM)),
            full((N, DIM)),
            vec, vec, vec, vec, vec, vec, vec, vec, vec,
            full((DIM, 1)),
        ],
        out_specs=full((BATCH * N, 1)),
        out_shape=jax.ShapeDtypeStruct((BATCH * N, 1), f32),
        scratch_shapes=[
            pltpu.VMEM((N, N), f32),
            pltpu.VMEM((N, 1), f32),
            pltpu.VMEM((1, N), f32),
            pltpu.VMEM((BATCH * N, DIM), f32),
            pltpu.VMEM((8, DIM), f32),
        ],
        interpret=_INTERPRET,
    )(data, lin_w, emb,
      att_i.reshape(1, DIM), att_j.reshape(1, DIM),
      att_em_i.reshape(1, DIM), att_em_j.reshape(1, DIM),
      gnn_bias.reshape(1, DIM), bn1_g.reshape(1, DIM), bn1_b.reshape(1, DIM),
      bn2_g.reshape(1, DIM), bn2_b.reshape(1, DIM), out_w)

    return (res + out_b).reshape(BATCH, N)


# speedup vs baseline: 1.1526x; 1.0408x over previous
import jax
import jax.numpy as jnp
from jax.experimental import pallas as pl
from jax.experimental.pallas import tpu as pltpu

BATCH = 32
N = 512
DIM = 128
INPUT_DIM = 64
TOPK = 20
EPS = 1e-5
NEG = -1e30
BSUB = 4
NSTEP = BATCH // BSUB

_INTERPRET = False


def _bf16_dot(a, b):
    return jax.lax.dot_general(
        a.astype(jnp.bfloat16), b.astype(jnp.bfloat16),
        (((1,), (0,)), ((), ())), preferred_element_type=jnp.float32)


def _split3_dot(a, b):
    a_hi = a.astype(jnp.bfloat16)
    a_lo = (a - a_hi.astype(jnp.float32)).astype(jnp.bfloat16)
    b_hi = b.astype(jnp.bfloat16)
    b_lo = (b - b_hi.astype(jnp.float32)).astype(jnp.bfloat16)
    dot = lambda p, q: jax.lax.dot_general(
        p, q, (((1,), (0,)), ((), ())), preferred_element_type=jnp.float32)
    return dot(a_hi, b_hi) + (dot(a_hi, b_lo) + dot(a_lo, b_hi))


def _fused_kernel(data_ref, lin_w_ref, emb_ref, att_i_ref, att_j_ref,
                  att_em_i_ref, att_em_j_ref, gnn_bias_ref, bn1_g_ref,
                  bn1_b_ref, bn2_g_ref, bn2_b_ref, out_w_ref,
                  res_ref,
                  bias_s, embi_s, embj_s, out_s, stats_s):
    s = pl.program_id(0)

    @pl.when(s == 0)
    def _graph():
        emb = emb_ref[:]
        embb = emb.astype(jnp.bfloat16)
        gram = jax.lax.dot_general(embb, embb, (((1,), (1,)), ((), ())),
                                   preferred_element_type=jnp.float32)
        nrm_col = jnp.sqrt(jnp.sum(emb * emb, axis=1, keepdims=True))
        cos = gram / (nrm_col * nrm_col.T)
        lane = jax.lax.broadcasted_iota(jnp.int32, (N, N), 1)
        sub = jax.lax.broadcasted_iota(jnp.int32, (N, N), 0)

        def body(_, carry):
            c, m = carry
            rowmax = jnp.max(c, axis=1, keepdims=True)
            ismax = c == rowmax
            idx = jnp.min(jnp.where(ismax, lane, N), axis=1, keepdims=True)
            onehot = lane == idx
            m = jnp.where(onehot, 0.0, m)
            c = jnp.where(onehot, -2.0, c)
            return c, m

        diag = sub == lane
        m0 = jnp.where(diag, 0.0, NEG)
        c0 = jnp.where(diag, -2.0, cos)
        _, m = jax.lax.fori_loop(0, TOPK - 1, body, (c0, m0))
        bias_s[:] = m
        embi_s[:] = jnp.sum(emb * att_em_i_ref[:], axis=1, keepdims=True)
        embj_s[:] = jnp.sum(emb * att_em_j_ref[:], axis=1, keepdims=True).T
        stats_s[:] = jnp.zeros((8, DIM), jnp.float32)

    @pl.when((s >= 1) & (s <= NSTEP))
    def _attn():
        acc0 = stats_s[0:1]
        acc1 = stats_s[1:2]
        for i in range(BSUB):
            xb = data_ref[i]
            xl = _bf16_dot(xb, lin_w_ref[:])
            ai_col = (jnp.sum(xl * att_i_ref[:], axis=1, keepdims=True)
                      + embi_s[:])
            aj_col = jnp.sum(xl * att_j_ref[:], axis=1, keepdims=True)
            aj_row = aj_col.T + embj_s[:]
            alpha = ai_col + aj_row
            alpha = jnp.maximum(alpha, 0.2 * alpha) + bias_s[:]
            zmax = jnp.max(ai_col, keepdims=True) + jnp.max(aj_row, keepdims=True)
            smax = jnp.maximum(zmax, 0.2 * zmax)
            e = jnp.exp(alpha - smax)
            den = jnp.sum(e, axis=1, keepdims=True)
            agg = _split3_dot(e, xl) * (1.0 / (den + 1e-16))
            o = agg + gnn_bias_ref[:]
            out_s[pl.ds((s - 1) * (BSUB * N) + i * N, N), :] = o
            acc0 = acc0 + jnp.sum(o, axis=0, keepdims=True)
            acc1 = acc1 + jnp.sum(o * o, axis=0, keepdims=True)
        stats_s[0:1] = acc0
        stats_s[1:2] = acc1

    @pl.when(s == NSTEP + 1)
    def _bnout():
        inv = 1.0 / (BATCH * N)
        mu = stats_s[0:1] * inv
        var = stats_s[1:2] * inv - mu * mu
        o = out_s[:]
        o = (o - mu) / jnp.sqrt(var + EPS) * bn1_g_ref[:] + bn1_b_ref[:]
        o = jnp.maximum(o, 0.0)
        o = (o.reshape(BATCH, N, DIM) * emb_ref[:][None]).reshape(BATCH * N, DIM)
        mu2 = jnp.sum(o, axis=0, keepdims=True) * inv
        var2 = jnp.sum(o * o, axis=0, keepdims=True) * inv - mu2 * mu2
        o = (o - mu2) / jnp.sqrt(var2 + EPS) * bn2_g_ref[:] + bn2_b_ref[:]
        o = jnp.maximum(o, 0.0)
        res_ref[:] = _bf16_dot(o, out_w_ref[:])


def kernel(data, org_edge_index, emb, lin_w, att_i, att_j, att_em_i, att_em_j,
           gnn_bias, bn1_g, bn1_b, bn2_g, bn2_b, out_w, out_b):
    del org_edge_index
    f32 = jnp.float32
    full = lambda shape: pl.BlockSpec(shape, lambda s: (0,) * len(shape))
    vec = full((1, DIM))
    res = pl.pallas_call(
        _fused_kernel,
        grid=(NSTEP + 2,),
        in_specs=[
            pl.BlockSpec((BSUB, N, INPUT_DIM),
                         lambda s: (jnp.clip(s - 1, 0, NSTEP - 1), 0, 0)),
            full((INPUT_DIM, DIM)),
            full((N, DIM)),
            vec, vec, vec, vec, vec, vec, vec, vec, vec,
            full((DIM, 1)),
        ],
        out_specs=full((BATCH * N, 1)),
        out_shape=jax.ShapeDtypeStruct((BATCH * N, 1), f32),
        scratch_shapes=[
            pltpu.VMEM((N, N), f32),
            pltpu.VMEM((N, 1), f32),
            pltpu.VMEM((1, N), f32),
            pltpu.VMEM((BATCH * N, DIM), f32),
            pltpu.VMEM((8, DIM), f32),
        ],
        interpret=_INTERPRET,
    )(data, lin_w, emb,
      att_i.reshape(1, DIM), att_j.reshape(1, DIM),
      att_em_i.reshape(1, DIM), att_em_j.reshape(1, DIM),
      gnn_bias.reshape(1, DIM), bn1_g.reshape(1, DIM), bn1_b.reshape(1, DIM),
      bn2_g.reshape(1, DIM), bn2_b.reshape(1, DIM), out_w)

    return (res + out_b).reshape(BATCH, N)


# speedup vs baseline: 1.1531x; 1.0004x over previous
import jax
import jax.numpy as jnp
from jax.experimental import pallas as pl
from jax.experimental.pallas import tpu as pltpu

BATCH = 32
N = 512
DIM = 128
INPUT_DIM = 64
TOPK = 20
EPS = 1e-5
NEG = -1e30
BSUB = 4
NSTEP = BATCH // BSUB


def _bf16_dot(a, b):
    return jax.lax.dot_general(
        a.astype(jnp.bfloat16), b.astype(jnp.bfloat16),
        (((1,), (0,)), ((), ())), preferred_element_type=jnp.float32)


def _split3_dot(a, b):
    a_hi = a.astype(jnp.bfloat16)
    a_lo = (a - a_hi.astype(jnp.float32)).astype(jnp.bfloat16)
    b_hi = b.astype(jnp.bfloat16)
    b_lo = (b - b_hi.astype(jnp.float32)).astype(jnp.bfloat16)
    dot = lambda p, q: jax.lax.dot_general(
        p, q, (((1,), (0,)), ((), ())), preferred_element_type=jnp.float32)
    return dot(a_hi, b_hi) + (dot(a_hi, b_lo) + dot(a_lo, b_hi))


def _fused_kernel(data_ref, lin_w_ref, emb_ref, att_i_ref, att_j_ref,
                  att_em_i_ref, att_em_j_ref, gnn_bias_ref, bn1_g_ref,
                  bn1_b_ref, bn2_g_ref, bn2_b_ref, out_w_ref,
                  res_ref,
                  bias_s, embi_s, embj_s, out_s, stats_s):
    s = pl.program_id(0)

    @pl.when(s == 0)
    def _graph():
        emb = emb_ref[:]
        embb = emb.astype(jnp.bfloat16)
        gram = jax.lax.dot_general(embb, embb, (((1,), (1,)), ((), ())),
                                   preferred_element_type=jnp.float32)
        nrm_col = jnp.sqrt(jnp.sum(emb * emb, axis=1, keepdims=True))
        cos = gram / (nrm_col * nrm_col.T)
        lane = jax.lax.broadcasted_iota(jnp.int32, (N, N), 1)
        sub = jax.lax.broadcasted_iota(jnp.int32, (N, N), 0)

        def body(_, carry):
            c, m = carry
            rowmax = jnp.max(c, axis=1, keepdims=True)
            ismax = c == rowmax
            idx = jnp.min(jnp.where(ismax, lane, N), axis=1, keepdims=True)
            onehot = lane == idx
            m = jnp.where(onehot, 0.0, m)
            c = jnp.where(onehot, -2.0, c)
            return c, m

        diag = sub == lane
        m0 = jnp.where(diag, 0.0, NEG)
        c0 = jnp.where(diag, -2.0, cos)
        _, m = jax.lax.fori_loop(0, TOPK - 1, body, (c0, m0))
        bias_s[:] = m
        embi_s[:] = jnp.sum(emb * att_em_i_ref[:], axis=1, keepdims=True)
        embj_s[:] = jnp.sum(emb * att_em_j_ref[:], axis=1, keepdims=True).T
        stats_s[:] = jnp.zeros((8, DIM), jnp.float32)

    @pl.when((s >= 1) & (s <= NSTEP))
    def _attn():
        acc0 = stats_s[0:1]
        acc1 = stats_s[1:2]
        for i in range(BSUB):
            xb = data_ref[i]
            xl = _bf16_dot(xb, lin_w_ref[:])
            ai_col = (jnp.sum(xl * att_i_ref[:], axis=1, keepdims=True)
                      + embi_s[:])
            aj_col = jnp.sum(xl * att_j_ref[:], axis=1, keepdims=True)
            aj_row = aj_col.T + embj_s[:]
            alpha = ai_col + aj_row
            alpha = jnp.maximum(alpha, 0.2 * alpha) + bias_s[:]
            zmax = jnp.max(ai_col, keepdims=True) + jnp.max(aj_row, keepdims=True)
            smax = jnp.maximum(zmax, 0.2 * zmax)
            e = jnp.exp(alpha - smax)
            den = jnp.sum(e, axis=1, keepdims=True)
            agg = _split3_dot(e, xl) * (1.0 / (den + 1e-16))
            o = agg + gnn_bias_ref[:]
            out_s[pl.ds((s - 1) * (BSUB * N) + i * N, N), :] = o
            acc0 = acc0 + jnp.sum(o, axis=0, keepdims=True)
            acc1 = acc1 + jnp.sum(o * o, axis=0, keepdims=True)
        stats_s[0:1] = acc0
        stats_s[1:2] = acc1

    @pl.when(s == NSTEP + 1)
    def _bnout():
        inv = 1.0 / (BATCH * N)
        mu = stats_s[0:1] * inv
        var = stats_s[1:2] * inv - mu * mu
        o = out_s[:]
        o = (o - mu) / jnp.sqrt(var + EPS) * bn1_g_ref[:] + bn1_b_ref[:]
        o = jnp.maximum(o, 0.0)
        o = (o.reshape(BATCH, N, DIM) * emb_ref[:][None]).reshape(BATCH * N, DIM)
        mu2 = jnp.sum(o, axis=0, keepdims=True) * inv
        var2 = jnp.sum(o * o, axis=0, keepdims=True) * inv - mu2 * mu2
        o = (o - mu2) / jnp.sqrt(var2 + EPS) * bn2_g_ref[:] + bn2_b_ref[:]
        o = jnp.maximum(o, 0.0)
        res_ref[:] = _bf16_dot(o, out_w_ref[:])


def kernel(data, org_edge_index, emb, lin_w, att_i, att_j, att_em_i, att_em_j,
           gnn_bias, bn1_g, bn1_b, bn2_g, bn2_b, out_w, out_b):
    del org_edge_index
    f32 = jnp.float32
    full = lambda shape: pl.BlockSpec(shape, lambda s: (0,) * len(shape))
    vec = full((1, DIM))
    res = pl.pallas_call(
        _fused_kernel,
        grid=(NSTEP + 2,),
        in_specs=[
            pl.BlockSpec((BSUB, N, INPUT_DIM),
                         lambda s: (jnp.clip(s - 1, 0, NSTEP - 1), 0, 0)),
            full((INPUT_DIM, DIM)),
            full((N, DIM)),
            vec, vec, vec, vec, vec, vec, vec, vec, vec,
            full((DIM, 1)),
        ],
        out_specs=full((BATCH * N, 1)),
        out_shape=jax.ShapeDtypeStruct((BATCH * N, 1), f32),
        scratch_shapes=[
            pltpu.VMEM((N, N), f32),
            pltpu.VMEM((N, 1), f32),
            pltpu.VMEM((1, N), f32),
            pltpu.VMEM((BATCH * N, DIM), f32),
            pltpu.VMEM((8, DIM), f32),
        ],
    )(data, lin_w, emb,
      att_i.reshape(1, DIM), att_j.reshape(1, DIM),
      att_em_i.reshape(1, DIM), att_em_j.reshape(1, DIM),
      gnn_bias.reshape(1, DIM), bn1_g.reshape(1, DIM), bn1_b.reshape(1, DIM),
      bn2_g.reshape(1, DIM), bn2_b.reshape(1, DIM), out_w)

    return (res + out_b).reshape(BATCH, N)
